# Initial kernel scaffold; baseline (speedup 1.0000x reference)
import jax
import jax.numpy as jnp
from jax.experimental import pallas as pl


def kernel(z, mask, emb):
    raise NotImplementedError("write your pallas kernel here")


# speedup vs baseline: 2.8811x; 2.8811x over previous
import functools

import jax
import jax.numpy as jnp
from jax.experimental import pallas as pl
from jax.experimental.pallas import tpu as pltpu

N_BATCH = 16
L = 1024
N_E = 1024
E_DIM = 64
BETA = 0.25
N_ROWS = N_BATCH * L


def _vq_kernel(zp_ref, mask_ref, emb_ref,
               zq_ref, enc_ref, idx_ref, loss_ref, perp_ref,
               cnt_ref, ssq_ref):
    b = pl.program_id(0)

    zp = zp_ref[...]
    emb = emb_ref[...]
    mask = mask_ref[...]

    zf2 = jnp.sum(zp * zp, axis=1, keepdims=True)
    emb2 = jnp.sum(emb * emb, axis=1)
    mm = jax.lax.dot_general(zp, emb, (((1,), (1,)), ((), ())),
                             preferred_element_type=jnp.float32)
    d = zf2 + emb2 - 2.0 * mm

    dmin = jnp.min(d, axis=1, keepdims=True)
    ii = jax.lax.broadcasted_iota(jnp.int32, (L, N_E), 1)
    idx = jnp.min(jnp.where(d == dmin, ii, jnp.int32(N_E)), axis=1,
                  keepdims=True)
    idx_ref[...] = idx

    onehot = (ii == idx).astype(jnp.float32)
    enc_ref[...] = onehot

    zq = jax.lax.dot_general(onehot, emb, (((1,), (0,)), ((), ())),
                             preferred_element_type=jnp.float32)
    diff = zq - zp
    zq_ref[...] = zp + diff

    masked = diff * mask
    sq = masked * masked

    @pl.when(b == 0)
    def _init():
        cnt_ref[...] = jnp.zeros_like(cnt_ref)
        ssq_ref[...] = jnp.zeros_like(ssq_ref)

    cnt_ref[...] += jnp.sum(onehot, axis=0, keepdims=True)
    ssq_ref[...] += jnp.sum(sq, axis=(0, 1), keepdims=True)

    @pl.when(b == N_BATCH - 1)
    def _finish():
        c = ssq_ref[...] / jnp.float32(N_ROWS * E_DIM)
        loss_ref[...] = c + jnp.float32(BETA) * c
        e_mean = cnt_ref[...] / jnp.float32(N_ROWS)
        ent = jnp.sum(e_mean * jnp.log(e_mean + 1e-10), axis=(0, 1),
                      keepdims=True)
        perp_ref[...] = jnp.exp(-ent)


@functools.partial(jax.jit, static_argnames=("interpret",))
def kernel(z, mask, emb, interpret=False):
    zp = jnp.transpose(z, (0, 2, 1)).reshape(N_ROWS, E_DIM)
    mask_col = mask.reshape(N_ROWS, 1)

    out_shape = [
        jax.ShapeDtypeStruct((N_ROWS, E_DIM), jnp.float32),
        jax.ShapeDtypeStruct((N_ROWS, N_E), jnp.float32),
        jax.ShapeDtypeStruct((N_ROWS, 1), jnp.int32),
        jax.ShapeDtypeStruct((1, 1), jnp.float32),
        jax.ShapeDtypeStruct((1, 1), jnp.float32),
    ]
    zq_rows, enc, idx, loss2, perp2 = pl.pallas_call(
        _vq_kernel,
        grid=(N_BATCH,),
        in_specs=[
            pl.BlockSpec((L, E_DIM), lambda b: (b, 0)),
            pl.BlockSpec((L, 1), lambda b: (b, 0)),
            pl.BlockSpec((N_E, E_DIM), lambda b: (0, 0)),
        ],
        out_specs=[
            pl.BlockSpec((L, E_DIM), lambda b: (b, 0)),
            pl.BlockSpec((L, N_E), lambda b: (b, 0)),
            pl.BlockSpec((L, 1), lambda b: (b, 0)),
            pl.BlockSpec((1, 1), lambda b: (0, 0)),
            pl.BlockSpec((1, 1), lambda b: (0, 0)),
        ],
        out_shape=out_shape,
        scratch_shapes=[
            pltpu.VMEM((1, N_E), jnp.float32),
            pltpu.VMEM((1, 1), jnp.float32),
        ],
        compiler_params=pltpu.CompilerParams(
            dimension_semantics=("arbitrary",)),
        interpret=interpret,
    )(zp, mask_col, emb)

    z_q_out = jnp.transpose(zq_rows.reshape(N_BATCH, L, E_DIM), (0, 2, 1))
    loss = loss2[0, 0]
    perplexity = perp2[0, 0]
    return (loss, z_q_out, perplexity, enc, idx)


# speedup vs baseline: 3.6611x; 1.2707x over previous
import functools

import jax
import jax.numpy as jnp
from jax.experimental import pallas as pl
from jax.experimental.pallas import tpu as pltpu

N_BATCH = 16
L = 1024
N_E = 1024
E_DIM = 64
BETA = 0.25
N_ROWS = N_BATCH * L


def _vq_kernel(z_ref, mask_ref, emb_ref,
               zq_ref, enc_ref, idx_ref, loss_ref, perp_ref,
               cnt_ref, ssq_ref, emb2_ref):
    b = pl.program_id(0)

    emb = emb_ref[...]

    @pl.when(b == 0)
    def _init():
        cnt_ref[...] = jnp.zeros_like(cnt_ref)
        ssq_ref[...] = jnp.zeros_like(ssq_ref)
        emb2_ref[...] = jnp.sum(emb * emb, axis=1, keepdims=True).T

    zp = jnp.transpose(z_ref[0], (1, 0))
    mask = mask_ref[...]

    zf2 = jnp.sum(zp * zp, axis=1, keepdims=True)
    emb2 = emb2_ref[...]
    mm = jax.lax.dot_general(zp, emb, (((1,), (1,)), ((), ())),
                             preferred_element_type=jnp.float32)
    d = zf2 + emb2 - 2.0 * mm

    dmin = jnp.min(d, axis=1, keepdims=True)
    ii = jax.lax.broadcasted_iota(jnp.int32, (L, N_E), 1)
    idx = jnp.min(jnp.where(d == dmin, ii, jnp.int32(N_E)), axis=1,
                  keepdims=True)
    idx_ref[...] = idx

    onehot = (ii == idx).astype(jnp.float32)
    enc_ref[...] = onehot

    zq = jax.lax.dot_general(onehot, emb, (((1,), (0,)), ((), ())),
                             preferred_element_type=jnp.float32)
    diff = zq - zp
    zq_ref[0] = jnp.transpose(zp + diff, (1, 0))

    masked = diff * mask
    sq = masked * masked

    ones_row = jnp.ones((1, L), jnp.float32)
    cnt_ref[...] += jax.lax.dot_general(
        ones_row, onehot, (((1,), (0,)), ((), ())),
        preferred_element_type=jnp.float32)
    ssq_ref[...] += jnp.sum(sq, axis=(0, 1), keepdims=True)

    @pl.when(b == N_BATCH - 1)
    def _finish():
        c = ssq_ref[...] / jnp.float32(N_ROWS * E_DIM)
        loss_ref[...] = c + jnp.float32(BETA) * c
        e_mean = cnt_ref[...] / jnp.float32(N_ROWS)
        ent = jnp.sum(e_mean * jnp.log(e_mean + 1e-10), axis=(0, 1),
                      keepdims=True)
        perp_ref[...] = jnp.exp(-ent)


@functools.partial(jax.jit, static_argnames=("interpret",))
def kernel(z, mask, emb, interpret=False):
    mask_col = mask.reshape(N_ROWS, 1)

    out_shape = [
        jax.ShapeDtypeStruct((N_BATCH, E_DIM, L), jnp.float32),
        jax.ShapeDtypeStruct((N_ROWS, N_E), jnp.float32),
        jax.ShapeDtypeStruct((N_ROWS, 1), jnp.int32),
        jax.ShapeDtypeStruct((1, 1), jnp.float32),
        jax.ShapeDtypeStruct((1, 1), jnp.float32),
    ]
    z_q_out, enc, idx, loss2, perp2 = pl.pallas_call(
        _vq_kernel,
        grid=(N_BATCH,),
        in_specs=[
            pl.BlockSpec((1, E_DIM, L), lambda b: (b, 0, 0)),
            pl.BlockSpec((L, 1), lambda b: (b, 0)),
            pl.BlockSpec((N_E, E_DIM), lambda b: (0, 0)),
        ],
        out_specs=[
            pl.BlockSpec((1, E_DIM, L), lambda b: (b, 0, 0)),
            pl.BlockSpec((L, N_E), lambda b: (b, 0)),
            pl.BlockSpec((L, 1), lambda b: (b, 0)),
            pl.BlockSpec((1, 1), lambda b: (0, 0)),
            pl.BlockSpec((1, 1), lambda b: (0, 0)),
        ],
        out_shape=out_shape,
        scratch_shapes=[
            pltpu.VMEM((1, N_E), jnp.float32),
            pltpu.VMEM((1, 1), jnp.float32),
            pltpu.VMEM((1, N_E), jnp.float32),
        ],
        compiler_params=pltpu.CompilerParams(
            dimension_semantics=("arbitrary",)),
        interpret=interpret,
    )(z, mask_col, emb)

    loss = loss2[0, 0]
    perplexity = perp2[0, 0]
    return (loss, z_q_out, perplexity, enc, idx)
